# Initial kernel scaffold; baseline (speedup 1.0000x reference)
import jax
import jax.numpy as jnp
from jax.experimental import pallas as pl


def kernel(H_d, H_t, A, W1_0, W2_0, Wg_0, W1_1, W2_1, Wg_1):
    raise NotImplementedError("write your pallas kernel here")


# speedup vs baseline: 17.4644x; 17.4644x over previous
import jax
import jax.numpy as jnp
from jax.experimental import pallas as pl
from jax.experimental.pallas import tpu as pltpu

N = 2048
TOPK = 32
BLK = 256
F32 = jnp.float32


HI = jax.lax.Precision.HIGHEST


def _proj_norm_body(h_ref, w_ref, o_ref):
    p = jnp.dot(h_ref[...], w_ref[...], preferred_element_type=F32)
    nrm = jnp.sqrt(jnp.sum(p * p, axis=1, keepdims=True))
    o_ref[...] = p / jnp.maximum(nrm, 1e-12)


def _proj_norm(h, w):
    m, f = h.shape
    u = w.shape[1]
    return pl.pallas_call(
        _proj_norm_body,
        grid=(m // BLK,),
        in_specs=[
            pl.BlockSpec((BLK, f), lambda i: (i, 0)),
            pl.BlockSpec((f, u), lambda i: (0, 0)),
        ],
        out_specs=pl.BlockSpec((BLK, u), lambda i: (i, 0)),
        out_shape=jax.ShapeDtypeStruct((m, u), F32),
    )(h, w)


def _sim_topk_body(hd_ref, ht_ref, bprev_ref, bout_ref, rs_ref, s_scr, w_scr):
    x = jax.lax.dot_general(hd_ref[...], ht_ref[...],
                            (((1,), (1,)), ((), ())),
                            preferred_element_type=F32)
    s_scr[...] = x
    w_scr[...] = x
    for _ in range(TOPK - 1):
        m = jnp.max(w_scr[...], axis=1, keepdims=True)
        w_scr[...] = jnp.where(w_scr[...] >= m, -3.0, w_scr[...])
    t = jnp.max(w_scr[...], axis=1, keepdims=True)
    x = s_scr[...]
    bnew = bprev_ref[...] + jnp.where(x >= t, jax.nn.sigmoid(x), 0.0)
    bout_ref[...] = bnew
    rs_ref[...] = jnp.sum(bnew, axis=1, keepdims=True)


def _sim_topk(hdp, htp, bprev):
    u = hdp.shape[1]
    return pl.pallas_call(
        _sim_topk_body,
        grid=(N // BLK,),
        in_specs=[
            pl.BlockSpec((BLK, u), lambda i: (i, 0)),
            pl.BlockSpec((N, u), lambda i: (0, 0)),
            pl.BlockSpec((BLK, N), lambda i: (i, 0)),
        ],
        out_specs=[
            pl.BlockSpec((BLK, N), lambda i: (i, 0)),
            pl.BlockSpec((BLK, 1), lambda i: (i, 0)),
        ],
        out_shape=[
            jax.ShapeDtypeStruct((N, N), F32),
            jax.ShapeDtypeStruct((N, 1), F32),
        ],
        scratch_shapes=[
            pltpu.VMEM((BLK, N), F32),
            pltpu.VMEM((BLK, N), F32),
        ],
    )(hdp, htp, bprev)


def _ptop_body(hd_ref, wg_ref, rs_ref, o_ref):
    dd = jax.lax.rsqrt(1.0 + rs_ref[...])
    o_ref[...] = dd * jnp.dot(hd_ref[...], wg_ref[...],
                              preferred_element_type=F32)


def _ptop(hd, wg, rs):
    f, u = wg.shape
    return pl.pallas_call(
        _ptop_body,
        grid=(N // BLK,),
        in_specs=[
            pl.BlockSpec((BLK, f), lambda i: (i, 0)),
            pl.BlockSpec((f, u), lambda i: (0, 0)),
            pl.BlockSpec((BLK, 1), lambda i: (i, 0)),
        ],
        out_specs=pl.BlockSpec((BLK, u), lambda i: (i, 0)),
        out_shape=jax.ShapeDtypeStruct((N, u), F32),
    )(hd, wg, rs)


def _bot_body(b_ref, ht_ref, wg_ref, ptop_ref, obot_ref, pbot_ref):
    b = b_ref[...]
    cs = jax.lax.dot_general(b, jnp.ones((N, 1), F32),
                             (((0,), (0,)), ((), ())),
                             preferred_element_type=F32)
    dt = jax.lax.rsqrt(1.0 + cs)
    mt = jnp.dot(ht_ref[...], wg_ref[...], preferred_element_type=F32)
    pbot = dt * mt
    pbot_ref[...] = pbot
    btp = jax.lax.dot_general(b, ptop_ref[...],
                              (((0,), (0,)), ((), ())),
                              preferred_element_type=F32)
    obot_ref[...] = jnp.maximum(dt * (pbot + btp), 0.0)


def _bot(bmat, ht, wg, ptop):
    f, u = wg.shape
    return pl.pallas_call(
        _bot_body,
        grid=(N // BLK,),
        in_specs=[
            pl.BlockSpec((N, BLK), lambda j: (0, j)),
            pl.BlockSpec((BLK, f), lambda j: (j, 0)),
            pl.BlockSpec((f, u), lambda j: (0, 0)),
            pl.BlockSpec((N, u), lambda j: (0, 0)),
        ],
        out_specs=[
            pl.BlockSpec((BLK, u), lambda j: (j, 0)),
            pl.BlockSpec((BLK, u), lambda j: (j, 0)),
        ],
        out_shape=[
            jax.ShapeDtypeStruct((N, u), F32),
            jax.ShapeDtypeStruct((N, u), F32),
        ],
    )(bmat, ht, wg, ptop)


def _top_body(b_ref, hd_ref, wg_ref, rs_ref, pbot_ref, otop_ref):
    dd = jax.lax.rsqrt(1.0 + rs_ref[...])
    md = jnp.dot(hd_ref[...], wg_ref[...], preferred_element_type=F32)
    acc = jnp.dot(b_ref[...], pbot_ref[...], preferred_element_type=F32)
    otop_ref[...] = jnp.maximum(dd * (dd * md + acc), 0.0)


def _top(bmat, hd, wg, rs, pbot):
    f, u = wg.shape
    return pl.pallas_call(
        _top_body,
        grid=(N // BLK,),
        in_specs=[
            pl.BlockSpec((BLK, N), lambda i: (i, 0)),
            pl.BlockSpec((BLK, f), lambda i: (i, 0)),
            pl.BlockSpec((f, u), lambda i: (0, 0)),
            pl.BlockSpec((BLK, 1), lambda i: (i, 0)),
            pl.BlockSpec((N, u), lambda i: (0, 0)),
        ],
        out_specs=pl.BlockSpec((BLK, u), lambda i: (i, 0)),
        out_shape=jax.ShapeDtypeStruct((N, u), F32),
    )(bmat, hd, wg, rs, pbot)


def _pred_body(hd_ref, ht_ref, o_ref):
    o_ref[...] = jax.lax.dot_general(hd_ref[...], ht_ref[...],
                                     (((1,), (1,)), ((), ())),
                                     preferred_element_type=F32)


def _pred(hd, ht):
    u = hd.shape[1]
    return pl.pallas_call(
        _pred_body,
        grid=(N // BLK,),
        in_specs=[
            pl.BlockSpec((BLK, u), lambda i: (i, 0)),
            pl.BlockSpec((N, u), lambda i: (0, 0)),
        ],
        out_specs=pl.BlockSpec((BLK, N), lambda i: (i, 0)),
        out_shape=jax.ShapeDtypeStruct((N, N), F32),
    )(hd, ht)


def kernel(H_d, H_t, A, W1_0, W2_0, Wg_0, W1_1, W2_1, Wg_1):
    bmat = A[:N, N:]
    for w1, w2, wg in ((W1_0, W2_0, Wg_0), (W1_1, W2_1, Wg_1)):
        hdp = _proj_norm(H_d, w1)
        htp = _proj_norm(H_t, w2)
        bmat, rs = _sim_topk(hdp, htp, bmat)
        ptop = _ptop(H_d, wg, rs)
        ht_new, pbot = _bot(bmat, H_t, wg, ptop)
        hd_new = _top(bmat, H_d, wg, rs, pbot)
        H_d, H_t = hd_new, ht_new
    r_pred = _pred(H_d, H_t)
    return (r_pred, H_d, H_t)
